# baseline (speedup 1.0000x reference)
import functools

import jax
import jax.numpy as jnp
from jax import lax
from jax.experimental import pallas as pl
from jax.experimental.pallas import tpu as pltpu
from jax.experimental.pallas import tpu_sc as plsc

_HIDDEN = 32
_BATCH = 4096
_T = 200
_NC = 2
_NS = 16
_NW = _NC * _NS
_LB = _BATCH // _NW
_RH = _HIDDEN // 8
_NPAIR = _T // 2


def _transpose_chunk(rows, outb, hv):
    i0, i1 = hv
    def grp(i, carry):
        for u in range(8):
            lb = i * 8 + u
            lbv = jnp.zeros((16,), jnp.int32) + lb
            v1 = rows[lb, pl.ds(0, 16)]
            v2 = rows[lb, pl.ds(16, 16)]
            plsc.store_scatter(outb, [i0[0], i1[0], lbv], v1)
            plsc.store_scatter(outb, [i0[1], i1[1], lbv], v2)
        return carry

    lax.fori_loop(0, _LB // 8, grp, 0)


def _emb_body(ids_hbm, table_hbm, out_hbm,
              idx_v, rows_a, rows_b, out_a, out_b,
              gsem_a, gsem_b, osem_a, osem_b):
    wid = lax.axis_index("s") * _NC + lax.axis_index("c")
    pltpu.sync_copy(ids_hbm.at[:, pl.ds(wid * _LB, _LB)], idx_v)

    iota = jnp.arange(16, dtype=jnp.int32)
    hv = ([iota // 8, (iota + 16) // 8], [iota % 8, (iota + 16) % 8])

    def fire_g(t, rows, sem):
        pltpu.make_async_copy(table_hbm.at[idx_v.at[t]], rows, sem).start()

    def wait_g(t, rows, sem):
        pltpu.make_async_copy(table_hbm.at[idx_v.at[t]], rows, sem).wait()

    def fire_s(t, outb, sem):
        pltpu.make_async_copy(
            outb.at[:, :, pl.ds(0, _LB)], out_hbm.at[t, :, wid], sem).start()

    def wait_s(t, outb, sem):
        pltpu.make_async_copy(
            outb.at[:, :, pl.ds(0, _LB)], out_hbm.at[t, :, wid], sem).wait()

    fire_g(0, rows_a, gsem_a)
    fire_g(1, rows_b, gsem_b)
    wait_g(0, rows_a, gsem_a)
    _transpose_chunk(rows_a, out_a, hv)
    fire_s(0, out_a, osem_a)
    fire_g(2, rows_a, gsem_a)
    wait_g(1, rows_b, gsem_b)
    _transpose_chunk(rows_b, out_b, hv)
    fire_s(1, out_b, osem_b)
    fire_g(3, rows_b, gsem_b)

    def pair(p, carry):
        te = 2 * p
        to = 2 * p + 1
        wait_g(te, rows_a, gsem_a)
        wait_s(te - 2, out_a, osem_a)
        _transpose_chunk(rows_a, out_a, hv)
        fire_s(te, out_a, osem_a)
        fire_g(te + 2, rows_a, gsem_a)
        wait_g(to, rows_b, gsem_b)
        wait_s(to - 2, out_b, osem_b)
        _transpose_chunk(rows_b, out_b, hv)
        fire_s(to, out_b, osem_b)
        fire_g(to + 2, rows_b, gsem_b)
        return carry

    lax.fori_loop(1, _NPAIR - 1, pair, 0, unroll=2)

    te, to = _T - 2, _T - 1
    wait_g(te, rows_a, gsem_a)
    wait_s(te - 2, out_a, osem_a)
    _transpose_chunk(rows_a, out_a, hv)
    fire_s(te, out_a, osem_a)
    wait_g(to, rows_b, gsem_b)
    wait_s(to - 2, out_b, osem_b)
    _transpose_chunk(rows_b, out_b, hv)
    fire_s(to, out_b, osem_b)
    wait_s(te, out_a, osem_a)
    wait_s(to, out_b, osem_b)


@jax.jit
def _run(ids_t, table):
    mesh = plsc.VectorSubcoreMesh(core_axis_name="c", subcore_axis_name="s")
    f = functools.partial(
        pl.kernel,
        mesh=mesh,
        compiler_params=pltpu.CompilerParams(
            use_tc_tiling_on_sc=False, needs_layout_passes=False),
        out_type=jax.ShapeDtypeStruct((_T, _RH, _NW, 8, _LB), jnp.float32),
        scratch_types=[
            pltpu.VMEM((_T, _LB), jnp.int32),
            pltpu.VMEM((_LB, _HIDDEN), jnp.float32),
            pltpu.VMEM((_LB, _HIDDEN), jnp.float32),
            pltpu.VMEM((_RH, 8, 129), jnp.float32),
            pltpu.VMEM((_RH, 8, 129), jnp.float32),
            pltpu.SemaphoreType.DMA,
            pltpu.SemaphoreType.DMA,
            pltpu.SemaphoreType.DMA,
            pltpu.SemaphoreType.DMA,
        ],
    )(_emb_body)
    return f(ids_t, table)


def kernel(input_ids, embed_table):
    ids_t = input_ids.T.astype(jnp.int32)
    raw = _run(ids_t, embed_table)
    return (raw.transpose(2, 4, 0, 1, 3)
            .reshape(_BATCH, _T, _HIDDEN))


# speedup vs baseline: 1.0202x; 1.0202x over previous
import functools

import jax
import jax.numpy as jnp
from jax import lax
from jax.experimental import pallas as pl
from jax.experimental.pallas import tpu as pltpu
from jax.experimental.pallas import tpu_sc as plsc

_HIDDEN = 32
_BATCH = 4096
_T = 200
_NC = 2
_NS = 16
_NW = _NC * _NS
_LB = _BATCH // _NW
_RH = _HIDDEN // 8
_NPH = _T // 2


def _transpose2(rows, outb, idx):
    qv, i1, i2 = idx
    def grp(i, carry):
        for u in range(8):
            lb = i * 8 + u
            lbv = jnp.zeros((16,), jnp.int32) + lb
            for q in (0, 1):
                v1 = rows[q * _LB + lb, pl.ds(0, 16)]
                v2 = rows[q * _LB + lb, pl.ds(16, 16)]
                plsc.store_scatter(outb, [qv[q], i1[0], i2[0], lbv], v1)
                plsc.store_scatter(outb, [qv[q], i1[1], i2[1], lbv], v2)
        return carry

    lax.fori_loop(0, _LB // 8, grp, 0)


def _emb_body(ids_hbm, table_hbm, out_hbm,
              idx_v, rows_a, rows_b, out_a, out_b,
              gsem_a, gsem_b, osem_a, osem_b):
    wid = lax.axis_index("s") * _NC + lax.axis_index("c")
    pltpu.sync_copy(ids_hbm.at[:, pl.ds(wid * _LB, _LB)], idx_v)

    iota = jnp.arange(16, dtype=jnp.int32)
    zeros = jnp.zeros((16,), jnp.int32)
    idx = ([zeros, zeros + 1],
           [iota // 8, (iota + 16) // 8],
           [iota % 8, (iota + 16) % 8])

    def fire_g(j, rows, sem):
        pltpu.make_async_copy(
            table_hbm.at[idx_v.at[2 * j]], rows.at[pl.ds(0, _LB)], sem
        ).start()
        pltpu.make_async_copy(
            table_hbm.at[idx_v.at[2 * j + 1]], rows.at[pl.ds(_LB, _LB)], sem
        ).start()

    def wait_g(j, rows, sem):
        pltpu.make_async_copy(
            table_hbm.at[idx_v.at[2 * j]], rows.at[pl.ds(0, _LB)], sem
        ).wait()
        pltpu.make_async_copy(
            table_hbm.at[idx_v.at[2 * j + 1]], rows.at[pl.ds(_LB, _LB)], sem
        ).wait()

    def fire_s(j, outb, sem):
        pltpu.make_async_copy(
            outb.at[:, :, :, pl.ds(0, _LB)], out_hbm.at[j, :, :, wid], sem
        ).start()

    def wait_s(j, outb, sem):
        pltpu.make_async_copy(
            outb.at[:, :, :, pl.ds(0, _LB)], out_hbm.at[j, :, :, wid], sem
        ).wait()

    fire_g(0, rows_a, gsem_a)
    fire_g(1, rows_b, gsem_b)
    wait_g(0, rows_a, gsem_a)
    _transpose2(rows_a, out_a, idx)
    fire_s(0, out_a, osem_a)
    fire_g(2, rows_a, gsem_a)
    wait_g(1, rows_b, gsem_b)
    _transpose2(rows_b, out_b, idx)
    fire_s(1, out_b, osem_b)
    fire_g(3, rows_b, gsem_b)

    def pair(p, carry):
        je = 2 * p
        jo = 2 * p + 1
        wait_g(je, rows_a, gsem_a)
        wait_s(je - 2, out_a, osem_a)
        _transpose2(rows_a, out_a, idx)
        fire_s(je, out_a, osem_a)
        fire_g(je + 2, rows_a, gsem_a)
        wait_g(jo, rows_b, gsem_b)
        wait_s(jo - 2, out_b, osem_b)
        _transpose2(rows_b, out_b, idx)
        fire_s(jo, out_b, osem_b)
        fire_g(jo + 2, rows_b, gsem_b)
        return carry

    lax.fori_loop(1, _NPH // 2 - 1, pair, 0)

    je, jo = _NPH - 2, _NPH - 1
    wait_g(je, rows_a, gsem_a)
    wait_s(je - 2, out_a, osem_a)
    _transpose2(rows_a, out_a, idx)
    fire_s(je, out_a, osem_a)
    wait_g(jo, rows_b, gsem_b)
    wait_s(jo - 2, out_b, osem_b)
    _transpose2(rows_b, out_b, idx)
    fire_s(jo, out_b, osem_b)
    wait_s(je, out_a, osem_a)
    wait_s(jo, out_b, osem_b)


@jax.jit
def _run(ids_t, table):
    mesh = plsc.VectorSubcoreMesh(core_axis_name="c", subcore_axis_name="s")
    f = functools.partial(
        pl.kernel,
        mesh=mesh,
        compiler_params=pltpu.CompilerParams(
            use_tc_tiling_on_sc=False, needs_layout_passes=False),
        out_type=jax.ShapeDtypeStruct((_NPH, 2, _RH, _NW, 8, _LB),
                                      jnp.float32),
        scratch_types=[
            pltpu.VMEM((_T, _LB), jnp.int32),
            pltpu.VMEM((2 * _LB, _HIDDEN), jnp.float32),
            pltpu.VMEM((2 * _LB, _HIDDEN), jnp.float32),
            pltpu.VMEM((2, _RH, 8, 129), jnp.float32),
            pltpu.VMEM((2, _RH, 8, 129), jnp.float32),
            pltpu.SemaphoreType.DMA,
            pltpu.SemaphoreType.DMA,
            pltpu.SemaphoreType.DMA,
            pltpu.SemaphoreType.DMA,
        ],
    )(_emb_body)
    return f(ids_t, table)


def kernel(input_ids, embed_table):
    ids_t = input_ids.T.astype(jnp.int32)
    raw = _run(ids_t, embed_table)
    return (raw.reshape(_T, _RH, _NW, 8, _LB)
            .transpose(2, 4, 0, 1, 3)
            .reshape(_BATCH, _T, _HIDDEN))
